# baseline (speedup 1.0000x reference)
import functools

import jax
import jax.numpy as jnp
from jax import lax
from jax.experimental import pallas as pl
from jax.experimental.pallas import tpu as pltpu
from jax.experimental.pallas import tpu_sc as plsc

_EMBED = 64
_NMETA = 16
_CTX = 128
_P = 50
_WIN = 128
_NW = 32
_BBLK = 256


def _sc_gather_combine(emb_table, flat_ids, meta):
    n = flat_ids.shape[1]
    bsz, p = meta.shape[0], meta.shape[1]
    nwin = n // _WIN
    wpw = nwin // _NW
    wpp = bsz // _WIN
    mesh = plsc.VectorSubcoreMesh(core_axis_name="c", subcore_axis_name="s")

    @functools.partial(
        pl.kernel,
        out_type=jax.ShapeDtypeStruct((n, _CTX), jnp.float32),
        mesh=mesh,
        compiler_params=pltpu.CompilerParams(use_tc_tiling_on_sc=False),
        scratch_types=[
            pltpu.VMEM((wpw * _WIN,), jnp.int32),
            pltpu.VMEM((_WIN, _EMBED), jnp.float32),
            pltpu.VMEM((_WIN, _EMBED), jnp.float32),
            pltpu.VMEM((_WIN, _CTX - 80), jnp.float32),
            pltpu.SemaphoreType.DMA,
            pltpu.SemaphoreType.DMA,
            pltpu.SemaphoreType.DMA,
            pltpu.SemaphoreType.DMA,
        ],
    )
    def k(tbl_hbm, idx_hbm, meta_hbm, out_hbm, idx_v, rows0, rows1, zeros_v,
          sidx, sg0, sg1, so):
        wid = lax.axis_index("s") * 2 + lax.axis_index("c")
        base_win = wid * wpw
        pltpu.async_copy(
            idx_hbm.at[0, pl.ds(base_win * _WIN, wpw * _WIN)], idx_v, sidx
        ).wait()
        zeros = jnp.zeros((16,), jnp.float32)

        @pl.loop(0, _WIN)
        def _(r):
            for c in range(0, _CTX - 80, 16):
                zeros_v[r, pl.ds(c, 16)] = zeros

        def issue(t, rows_v, sg):
            w = base_win + t
            p_ = w // wpp
            b0 = (w % wpp) * _WIN
            r0 = w * _WIN
            idx_slice = idx_v.at[pl.ds(t * _WIN, _WIN)]
            pltpu.async_copy(tbl_hbm.at[idx_slice], rows_v, sg).wait()
            emb_out = pltpu.async_copy(
                rows_v, out_hbm.at[pl.ds(r0, _WIN), pl.ds(0, _EMBED)], sg)
            meta_out = pltpu.async_copy(
                meta_hbm.at[pl.ds(b0, _WIN), p_, :],
                out_hbm.at[pl.ds(r0, _WIN), pl.ds(_EMBED, _NMETA)], sg)
            zero_out = pltpu.async_copy(
                zeros_v, out_hbm.at[pl.ds(r0, _WIN), pl.ds(80, _CTX - 80)], sg)
            return emb_out, meta_out, zero_out

        @pl.loop(0, wpw)
        def _(t):
            for c in issue(t, rows0, sg0):
                c.wait()

    return k(emb_table, flat_ids, meta)


def _tc_mlp2_body(comb_ref, w1p_ref, b1_ref, w2_ref, b2_ref,
                  gamma_ref, beta_ref, out_ref):
    p, nb = comb_ref.shape[0], comb_ref.shape[1]
    x = comb_ref[...].reshape(p * nb, _CTX)
    h = jnp.dot(x, w1p_ref[...], preferred_element_type=jnp.float32) + b1_ref[...]
    h = jnp.maximum(h, 0.0)
    h = jnp.dot(h, w2_ref[...], preferred_element_type=jnp.float32) + b2_ref[...]
    mean = jnp.mean(h, axis=-1, keepdims=True)
    d = h - mean
    var = jnp.mean(d * d, axis=-1, keepdims=True)
    y = d * jax.lax.rsqrt(var + 1e-5) * gamma_ref[...] + beta_ref[...]
    out_ref[...] = jnp.max(y.reshape(p, nb, _CTX), axis=0)


def _tc_mlp2(comb3, w1p, b1, w2, b2, gamma, beta):
    p, bsz = comb3.shape[0], comb3.shape[1]
    fixed = lambda i: (0, 0)
    return pl.pallas_call(
        _tc_mlp2_body,
        grid=(bsz // _BBLK,),
        in_specs=[
            pl.BlockSpec((p, _BBLK, _CTX), lambda i: (0, i, 0)),
            pl.BlockSpec((_CTX, _CTX), fixed),
            pl.BlockSpec((1, _CTX), fixed),
            pl.BlockSpec((_CTX, _CTX), fixed),
            pl.BlockSpec((1, _CTX), fixed),
            pl.BlockSpec((1, _CTX), fixed),
            pl.BlockSpec((1, _CTX), fixed),
        ],
        out_specs=pl.BlockSpec((_BBLK, _CTX), lambda i: (i, 0)),
        out_shape=jax.ShapeDtypeStruct((bsz, _CTX), jnp.float32),
    )(comb3, w1p, b1, w2, b2, gamma, beta)


def kernel(pattern_ids, pattern_metadata, emb_table, W1, b1, W2, b2, gamma, beta):
    bsz, p = pattern_ids.shape
    n = bsz * p
    flat_ids = pattern_ids.T.reshape(1, n).astype(jnp.int32)
    comb = _sc_gather_combine(emb_table, flat_ids, pattern_metadata)
    comb3 = comb.reshape(p, bsz, _CTX)
    w1p = jnp.concatenate([W1, jnp.zeros((_CTX - W1.shape[0], _CTX), W1.dtype)], axis=0)
    return _tc_mlp2(
        comb3, w1p, b1.reshape(1, _CTX), W2, b2.reshape(1, _CTX),
        gamma.reshape(1, _CTX), beta.reshape(1, _CTX),
    )
